# baseline (speedup 1.0000x reference)
import jax
import jax.numpy as jnp
from jax import lax
from jax.experimental import pallas as pl

NB, NT, NF, NMIC = 8, 100, 128, 2
NELE = NAZI = 73
NG = NELE * NAZI
D = NF * NMIC
BT = NB * NT
SCALE = (NMIC * NF) / 2.0
CHUNK = 160
NC = BT // CHUNK
NGP = 5376


def _argmax_rows(m):
    tmax = jnp.max(m, axis=1, keepdims=True)
    iota = lax.broadcasted_iota(jnp.int32, m.shape, 1)
    idx = jnp.min(jnp.where(m == tmax, iota, NG), axis=1, keepdims=True)
    return tmax, idx


def _gather_rows(idx, t):
    iota = lax.broadcasted_iota(jnp.int32, (CHUNK, NGP), 1)
    oh = jnp.where(iota == idx, 1.0, 0.0)
    return lax.dot_general(
        oh, t, (((1,), (0,)), ((), ())),
        preferred_element_type=jnp.float32,
        precision=lax.Precision.HIGHEST,
    )


def _body(ipd_ref, t_ref, doa_ref, ss_ref, doa4_ref, vad_ref):
    ip = ipd_ref[...]
    t = t_ref[...]

    m0 = lax.dot_general(
        ip, t, (((1,), (1,)), ((), ())),
        preferred_element_type=jnp.float32,
    )[:, :NG] * (1.0 / SCALE)
    ss_ref[...] = m0
    _, idx0 = _argmax_rows(m0)

    tm0 = _gather_rows(idx0, t)
    num0 = jnp.sum(tm0 * ip, axis=1, keepdims=True)
    den0 = jnp.sum(tm0 * tm0, axis=1, keepdims=True)
    r0 = num0 / den0
    cur1 = ip - r0 * tm0

    m1 = lax.dot_general(
        cur1, t, (((1,), (1,)), ((), ())),
        preferred_element_type=jnp.float32,
    )[:, :NG]
    _, idx1 = _argmax_rows(m1)

    tm1 = _gather_rows(idx1, t)
    num1 = jnp.sum(tm1 * cur1, axis=1, keepdims=True)
    den1 = jnp.sum(tm1 * tm1, axis=1, keepdims=True)
    r1 = num1 / den1

    vad_ref[...] = jnp.concatenate([r0, r1], axis=1)

    col = lax.broadcasted_iota(jnp.int32, (CHUNK, NAZI), 1)
    ele = doa_ref[0:1, :]
    azi = doa_ref[1:2, :]
    e0 = jnp.sum(jnp.where(col == idx0 // NAZI, ele, 0.0), axis=1,
                 keepdims=True)
    e1 = jnp.sum(jnp.where(col == idx1 // NAZI, ele, 0.0), axis=1,
                 keepdims=True)
    a0 = jnp.sum(jnp.where(col == idx0 % NAZI, azi, 0.0), axis=1,
                 keepdims=True)
    a1 = jnp.sum(jnp.where(col == idx1 % NAZI, azi, 0.0), axis=1,
                 keepdims=True)
    doa4_ref[...] = jnp.concatenate([e0, e1, a0, a1], axis=1)


def _pipeline(ipd, T, doa_candidate):
    return pl.pallas_call(
        _body,
        grid=(NC,),
        in_specs=[
            pl.BlockSpec((CHUNK, D), lambda i: (i, 0)),
            pl.BlockSpec((NGP, D), lambda i: (0, 0)),
            pl.BlockSpec((2, NAZI), lambda i: (0, 0)),
        ],
        out_specs=[
            pl.BlockSpec((CHUNK, NG), lambda i: (i, 0)),
            pl.BlockSpec((CHUNK, 4), lambda i: (i, 0)),
            pl.BlockSpec((CHUNK, 2), lambda i: (i, 0)),
        ],
        out_shape=[
            jax.ShapeDtypeStruct((BT, NG), jnp.float32),
            jax.ShapeDtypeStruct((BT, 4), jnp.float32),
            jax.ShapeDtypeStruct((BT, 2), jnp.float32),
        ],
    )(ipd, T, doa_candidate)


def kernel(pred_ipd, dpipd_template, doa_candidate):
    pred_ipd = lax.stop_gradient(pred_ipd)
    ipd = pred_ipd.reshape(BT, D)
    T = jnp.pad(dpipd_template.reshape(NG, D), ((0, NGP - NG), (0, 0)))
    ss, doa4, vad2 = _pipeline(ipd, T, doa_candidate)
    pred_ss = ss.reshape(NB, NT, NELE, NAZI)
    pred_DOAs = doa4.reshape(NB, NT, 2, 2)
    pred_VADs = vad2.reshape(NB, NT, 2)
    return (pred_DOAs, pred_VADs, pred_ss)


# speedup vs baseline: 1.1206x; 1.1206x over previous
import jax
import jax.numpy as jnp
from jax import lax
from jax.experimental import pallas as pl

NB, NT, NF, NMIC = 8, 100, 128, 2
NELE = NAZI = 73
NG = NELE * NAZI
D = NF * NMIC
BT = NB * NT
SCALE = (NMIC * NF) / 2.0
CHUNK = 160
NC = BT // CHUNK
NGP = 5376


def _argmax_rows(m):
    tmax = jnp.max(m, axis=1, keepdims=True)
    iota = lax.broadcasted_iota(jnp.int32, m.shape, 1)
    idx = jnp.min(jnp.where(m == tmax, iota, NG), axis=1, keepdims=True)
    return tmax, idx


def _gather_rows(idx, th, tm, tl):
    iota = lax.broadcasted_iota(jnp.int32, (CHUNK, NGP), 1)
    oh = jnp.where(iota == idx, 1.0, 0.0).astype(jnp.bfloat16)
    dims = (((1,), (0,)), ((), ()))
    parts = [
        lax.dot_general(oh, p, dims, preferred_element_type=jnp.float32)
        for p in (th, tm, tl)
    ]
    return (parts[0] + parts[1]) + parts[2]


def _body(ipd_ref, t_ref, th_ref, tm_ref, tl_ref, doa_ref,
          ss_ref, doa4_ref, vad_ref):
    ip = ipd_ref[...]
    t = t_ref[...]
    th = th_ref[...]
    tmid = tm_ref[...]
    tl = tl_ref[...]

    m0 = lax.dot_general(
        ip, t, (((1,), (1,)), ((), ())),
        preferred_element_type=jnp.float32,
    )[:, :NG] * (1.0 / SCALE)
    ss_ref[...] = m0
    _, idx0 = _argmax_rows(m0)

    tm0 = _gather_rows(idx0, th, tmid, tl)
    num0 = jnp.sum(tm0 * ip, axis=1, keepdims=True)
    den0 = jnp.sum(tm0 * tm0, axis=1, keepdims=True)
    r0 = num0 / den0
    cur1 = ip - r0 * tm0

    m1 = lax.dot_general(
        cur1, t, (((1,), (1,)), ((), ())),
        preferred_element_type=jnp.float32,
    )[:, :NG]
    _, idx1 = _argmax_rows(m1)

    tm1 = _gather_rows(idx1, th, tmid, tl)
    num1 = jnp.sum(tm1 * cur1, axis=1, keepdims=True)
    den1 = jnp.sum(tm1 * tm1, axis=1, keepdims=True)
    r1 = num1 / den1

    vad_ref[...] = jnp.concatenate([r0, r1], axis=1)

    col = lax.broadcasted_iota(jnp.int32, (CHUNK, NAZI), 1)
    ele = doa_ref[0:1, :]
    azi = doa_ref[1:2, :]
    e0 = jnp.sum(jnp.where(col == idx0 // NAZI, ele, 0.0), axis=1,
                 keepdims=True)
    e1 = jnp.sum(jnp.where(col == idx1 // NAZI, ele, 0.0), axis=1,
                 keepdims=True)
    a0 = jnp.sum(jnp.where(col == idx0 % NAZI, azi, 0.0), axis=1,
                 keepdims=True)
    a1 = jnp.sum(jnp.where(col == idx1 % NAZI, azi, 0.0), axis=1,
                 keepdims=True)
    doa4_ref[...] = jnp.concatenate([e0, e1, a0, a1], axis=1)


def _pipeline(ipd, T, Th, Tm, Tl, doa_candidate):
    return pl.pallas_call(
        _body,
        grid=(NC,),
        in_specs=[
            pl.BlockSpec((CHUNK, D), lambda i: (i, 0)),
            pl.BlockSpec((NGP, D), lambda i: (0, 0)),
            pl.BlockSpec((NGP, D), lambda i: (0, 0)),
            pl.BlockSpec((NGP, D), lambda i: (0, 0)),
            pl.BlockSpec((NGP, D), lambda i: (0, 0)),
            pl.BlockSpec((2, NAZI), lambda i: (0, 0)),
        ],
        out_specs=[
            pl.BlockSpec((CHUNK, NG), lambda i: (i, 0)),
            pl.BlockSpec((CHUNK, 4), lambda i: (i, 0)),
            pl.BlockSpec((CHUNK, 2), lambda i: (i, 0)),
        ],
        out_shape=[
            jax.ShapeDtypeStruct((BT, NG), jnp.float32),
            jax.ShapeDtypeStruct((BT, 4), jnp.float32),
            jax.ShapeDtypeStruct((BT, 2), jnp.float32),
        ],
    )(ipd, T, Th, Tm, Tl, doa_candidate)


def kernel(pred_ipd, dpipd_template, doa_candidate):
    pred_ipd = lax.stop_gradient(pred_ipd)
    ipd = pred_ipd.reshape(BT, D)
    T = jnp.pad(dpipd_template.reshape(NG, D), ((0, NGP - NG), (0, 0)))
    Th = T.astype(jnp.bfloat16)
    r1 = T - Th.astype(jnp.float32)
    Tm = r1.astype(jnp.bfloat16)
    Tl = (r1 - Tm.astype(jnp.float32)).astype(jnp.bfloat16)
    ss, doa4, vad2 = _pipeline(ipd, T, Th, Tm, Tl, doa_candidate)
    pred_ss = ss.reshape(NB, NT, NELE, NAZI)
    pred_DOAs = doa4.reshape(NB, NT, 2, 2)
    pred_VADs = vad2.reshape(NB, NT, 2)
    return (pred_DOAs, pred_VADs, pred_ss)


# speedup vs baseline: 1.1765x; 1.0500x over previous
import jax
import jax.numpy as jnp
from jax import lax
from jax.experimental import pallas as pl

NB, NT, NF, NMIC = 8, 100, 128, 2
NELE = NAZI = 73
NG = NELE * NAZI
D = NF * NMIC
BT = NB * NT
SCALE = (NMIC * NF) / 2.0
CHUNK = 160
NC = BT // CHUNK
NGP = 5376


def _argmax_rows(m):
    tmax = jnp.max(m, axis=1, keepdims=True)
    iota = lax.broadcasted_iota(jnp.int32, m.shape, 1)
    idx = jnp.min(jnp.where(m == tmax, iota, NG), axis=1, keepdims=True)
    return tmax, idx


def _gather_rows(idx, th, tm, tl):
    iota = lax.broadcasted_iota(jnp.int32, (CHUNK, NGP), 1)
    oh = jnp.where(iota == idx, 1.0, 0.0).astype(jnp.bfloat16)
    dims = (((1,), (0,)), ((), ()))
    parts = [
        lax.dot_general(oh, p, dims, preferred_element_type=jnp.float32)
        for p in (th, tm, tl)
    ]
    return (parts[0] + parts[1]) + parts[2]


def _body(ipd_ref, t_ref, doa_ref, ss_ref, doa4_ref, vad_ref,
          th_ref, tm_ref, tl_ref):
    ip = ipd_ref[...]
    t = t_ref[...]

    @pl.when(pl.program_id(0) == 0)
    def _():
        th = t.astype(jnp.bfloat16)
        r1 = t - th.astype(jnp.float32)
        tmid = r1.astype(jnp.bfloat16)
        tl = (r1 - tmid.astype(jnp.float32)).astype(jnp.bfloat16)
        th_ref[...] = th
        tm_ref[...] = tmid
        tl_ref[...] = tl

    th = th_ref[...]
    tmid = tm_ref[...]
    tl = tl_ref[...]

    m0 = lax.dot_general(
        ip, t, (((1,), (1,)), ((), ())),
        preferred_element_type=jnp.float32,
    )[:, :NG] * (1.0 / SCALE)
    ss_ref[...] = m0
    _, idx0 = _argmax_rows(m0)

    tm0 = _gather_rows(idx0, th, tmid, tl)
    num0 = jnp.sum(tm0 * ip, axis=1, keepdims=True)
    den0 = jnp.sum(tm0 * tm0, axis=1, keepdims=True)
    r0 = num0 / den0
    cur1 = ip - r0 * tm0

    m1 = lax.dot_general(
        cur1, t, (((1,), (1,)), ((), ())),
        preferred_element_type=jnp.float32,
    )[:, :NG]
    _, idx1 = _argmax_rows(m1)

    tm1 = _gather_rows(idx1, th, tmid, tl)
    num1 = jnp.sum(tm1 * cur1, axis=1, keepdims=True)
    den1 = jnp.sum(tm1 * tm1, axis=1, keepdims=True)
    r1 = num1 / den1

    vad_ref[...] = jnp.concatenate([r0, r1], axis=1)

    col = lax.broadcasted_iota(jnp.int32, (CHUNK, NAZI), 1)
    ele = doa_ref[0:1, :]
    azi = doa_ref[1:2, :]
    e0 = jnp.sum(jnp.where(col == idx0 // NAZI, ele, 0.0), axis=1,
                 keepdims=True)
    e1 = jnp.sum(jnp.where(col == idx1 // NAZI, ele, 0.0), axis=1,
                 keepdims=True)
    a0 = jnp.sum(jnp.where(col == idx0 % NAZI, azi, 0.0), axis=1,
                 keepdims=True)
    a1 = jnp.sum(jnp.where(col == idx1 % NAZI, azi, 0.0), axis=1,
                 keepdims=True)
    doa4_ref[...] = jnp.concatenate([e0, e1, a0, a1], axis=1)


def _pipeline(ipd, T, doa_candidate):
    from jax.experimental.pallas import tpu as pltpu

    return pl.pallas_call(
        _body,
        grid=(NC,),
        in_specs=[
            pl.BlockSpec((CHUNK, D), lambda i: (i, 0)),
            pl.BlockSpec((NGP, D), lambda i: (0, 0)),
            pl.BlockSpec((2, NAZI), lambda i: (0, 0)),
        ],
        out_specs=[
            pl.BlockSpec((CHUNK, NG), lambda i: (i, 0)),
            pl.BlockSpec((CHUNK, 4), lambda i: (i, 0)),
            pl.BlockSpec((CHUNK, 2), lambda i: (i, 0)),
        ],
        out_shape=[
            jax.ShapeDtypeStruct((BT, NG), jnp.float32),
            jax.ShapeDtypeStruct((BT, 4), jnp.float32),
            jax.ShapeDtypeStruct((BT, 2), jnp.float32),
        ],
        scratch_shapes=[
            pltpu.VMEM((NGP, D), jnp.bfloat16),
            pltpu.VMEM((NGP, D), jnp.bfloat16),
            pltpu.VMEM((NGP, D), jnp.bfloat16),
        ],
    )(ipd, T, doa_candidate)


def kernel(pred_ipd, dpipd_template, doa_candidate):
    pred_ipd = lax.stop_gradient(pred_ipd)
    ipd = pred_ipd.reshape(BT, D)
    T = jnp.pad(dpipd_template.reshape(NG, D), ((0, NGP - NG), (0, 0)))
    ss, doa4, vad2 = _pipeline(ipd, T, doa_candidate)
    pred_ss = ss.reshape(NB, NT, NELE, NAZI)
    pred_DOAs = doa4.reshape(NB, NT, 2, 2)
    pred_VADs = vad2.reshape(NB, NT, 2)
    return (pred_DOAs, pred_VADs, pred_ss)
